# baseline (speedup 1.0000x reference)
import functools

import jax
import jax.numpy as jnp
from jax import lax
from jax.experimental import pallas as pl
from jax.experimental.pallas import tpu as pltpu
from jax.experimental.pallas import tpu_sc as plsc

N = 10000
E = 160000
D = 256
DH = 128
EBLK = 128
NBLKS = E // EBLK
GBLK = 128
EPAD = 163840
NB = EPAD // GBLK // 16
NR = 2
NX = 4
NC, NS = 2, 16
HIST_N = 10240
HSLC = HIST_N // NS
RPT = 632
RPT_LAST = N - (NS - 1) * RPT
R = 1000


def _sc_mesh():
    return plsc.VectorSubcoreMesh(core_axis_name="c", subcore_axis_name="s")


def _deg_body(edge_hbm, deg0_hbm, deg1_hbm, dst_v, ones_v, zeros_v, hist_sh):
    c = lax.axis_index("c")
    s = lax.axis_index("s")

    for j in range(EBLK // 16):
        ones_v[pl.ds(j * 16, 16)] = jnp.ones((16,), jnp.float32)
    for j in range(HSLC // 16):
        zeros_v[pl.ds(j * 16, 16)] = jnp.zeros((16,), jnp.float32)

    pltpu.sync_copy(zeros_v, hist_sh.at[pl.ds(s * HSLC, HSLC)])
    plsc.subcore_barrier()

    w = c * NS + s

    @pl.loop(0, (NBLKS + NC * NS - 1) // (NC * NS))
    def _edge_blocks(i):
        bi = w + i * NC * NS

        @pl.when(bi < NBLKS)
        def _():
            pltpu.sync_copy(edge_hbm.at[1, pl.ds(bi * EBLK, EBLK)], dst_v)
            pltpu.sync_copy(ones_v, hist_sh.at[dst_v], add=True)

    plsc.subcore_barrier()

    @pl.when(c == 0)
    def _():
        pltpu.sync_copy(hist_sh.at[pl.ds(s * HSLC, HSLC)],
                        deg0_hbm.at[pl.ds(s * HSLC, HSLC)])

    @pl.when(c == 1)
    def _():
        pltpu.sync_copy(hist_sh.at[pl.ds(s * HSLC, HSLC)],
                        deg1_hbm.at[pl.ds(s * HSLC, HSLC)])


_deg_call = pl.kernel(
    _deg_body,
    out_type=(jax.ShapeDtypeStruct((HIST_N,), jnp.float32),
              jax.ShapeDtypeStruct((HIST_N,), jnp.float32)),
    mesh=_sc_mesh(),
    scratch_types=[
        pltpu.VMEM((EBLK,), jnp.int32),
        pltpu.VMEM((EBLK,), jnp.float32),
        pltpu.VMEM((HSLC,), jnp.float32),
        pltpu.VMEM_SHARED((HIST_N,), jnp.float32),
    ],
)


def _mm_body(d0_ref, d1_ref, x_ref, w_ref, g0_ref, g1_ref):
    deg = d0_ref[...] + d1_ref[...] + 1.0
    dinv = lax.rsqrt(deg)
    h = jnp.dot(x_ref[...], w_ref[...], preferred_element_type=jnp.float32)
    g = h * dinv
    g0_ref[...] = g[:, :DH]
    g1_ref[...] = g[:, DH:]


_mm_call = pl.pallas_call(
    _mm_body,
    grid=(N // R,),
    in_specs=[
        pl.BlockSpec((R, 1), lambda i: (i, 0)),
        pl.BlockSpec((R, 1), lambda i: (i, 0)),
        pl.BlockSpec((R, D), lambda i: (i, 0)),
        pl.BlockSpec((D, D), lambda i: (0, 0)),
    ],
    out_specs=[
        pl.BlockSpec((R, DH), lambda i: (i, 0)),
        pl.BlockSpec((R, DH), lambda i: (i, 0)),
    ],
    out_shape=[
        jax.ShapeDtypeStruct((N, DH), jnp.float32),
        jax.ShapeDtypeStruct((N, DH), jnp.float32),
    ],
)


def _edge_body(*a):
    g0_hbm, g1_hbm, src1d, dst1d, a0_hbm, a1_hbm = a[:6]
    sb = a[6:6 + NX]
    db = a[6 + NX:6 + 2 * NX]
    rows = a[6 + 2 * NX:6 + 2 * NX + NR]
    isem, xsem, gsem, ssem, acc_sh = a[6 + 2 * NX + NR:]

    c = lax.axis_index("c")
    s = lax.axis_index("s")

    def idx_copy(base, j, slot, make_only):
        mk = pltpu.make_async_copy if make_only else \
            lambda sr, dr, sm: pltpu.async_copy(sr, dr, sm)
        ds_ = pl.ds((base + j) * GBLK, GBLK)
        return (mk(src1d.at[ds_], sb[slot], xsem.at[slot]),
                mk(dst1d.at[ds_], db[slot], xsem.at[slot]))

    def work(g_hbm, o_hbm):
        base = s * NB

        @pl.when(s < NS - 1)
        def _():
            pltpu.async_copy(g_hbm.at[pl.ds(s * RPT, RPT)],
                             acc_sh.at[pl.ds(s * RPT, RPT)], isem)

        @pl.when(s == NS - 1)
        def _():
            pltpu.async_copy(g_hbm.at[pl.ds((NS - 1) * RPT, RPT_LAST)],
                             acc_sh.at[pl.ds((NS - 1) * RPT, RPT_LAST)], isem)

        for j in range(2):
            idx_copy(base, j, j, False)
        d1, d2 = idx_copy(base, 0, 0, True)
        d1.wait()
        d2.wait()
        pltpu.async_copy(g_hbm.at[sb[0]], rows[0], gsem.at[0])

        @pl.when(s < NS - 1)
        def _():
            pltpu.make_async_copy(g_hbm.at[pl.ds(s * RPT, RPT)],
                                  acc_sh.at[pl.ds(s * RPT, RPT)], isem).wait()

        @pl.when(s == NS - 1)
        def _():
            pltpu.make_async_copy(
                g_hbm.at[pl.ds((NS - 1) * RPT, RPT_LAST)],
                acc_sh.at[pl.ds((NS - 1) * RPT, RPT_LAST)], isem).wait()

        plsc.subcore_barrier()

        @pl.loop(0, NB, step=NX)
        def _blocks(o):
            for bs in range(NX):
                i = o + bs
                br = bs % NR
                br1, x1 = (bs + 1) % NR, (bs + 1) % NX
                x2 = (bs + 2) % NX

                pltpu.make_async_copy(g_hbm.at[sb[bs]], rows[br],
                                      gsem.at[br]).wait()
                pltpu.async_copy(rows[br], acc_sh.at[db[bs]],
                                 ssem.at[br], add=True)

                @pl.when(i + 1 < NB)
                def _():
                    d1, d2 = idx_copy(base, i + 1, x1, True)
                    d1.wait()
                    d2.wait()

                    @pl.when(i >= 1)
                    def _():
                        pltpu.make_async_copy(
                            rows[br1], acc_sh.at[db[x1]],
                            ssem.at[br1]).wait()

                    pltpu.async_copy(g_hbm.at[sb[x1]], rows[br1],
                                     gsem.at[br1])

                @pl.when(i + 2 < NB)
                def _():
                    idx_copy(base, i + 2, x2, False)

        for k in range(NR):
            pltpu.make_async_copy(rows[k], acc_sh.at[db[(NR + k) % NX]],
                                  ssem.at[k]).wait()

        plsc.subcore_barrier()

        @pl.when(s < NS - 1)
        def _():
            pltpu.sync_copy(acc_sh.at[pl.ds(s * RPT, RPT)],
                            o_hbm.at[pl.ds(s * RPT, RPT)])

        @pl.when(s == NS - 1)
        def _():
            pltpu.sync_copy(acc_sh.at[pl.ds((NS - 1) * RPT, RPT_LAST)],
                            o_hbm.at[pl.ds((NS - 1) * RPT, RPT_LAST)])

    @pl.when(c == 0)
    def _():
        work(g0_hbm, a0_hbm)

    @pl.when(c == 1)
    def _():
        work(g1_hbm, a1_hbm)


_edge_call = pl.kernel(
    _edge_body,
    out_type=(jax.ShapeDtypeStruct((N, DH), jnp.float32),
              jax.ShapeDtypeStruct((N, DH), jnp.float32)),
    mesh=_sc_mesh(),
    scratch_types=(
        [pltpu.VMEM((GBLK,), jnp.int32) for _ in range(2 * NX)]
        + [pltpu.VMEM((GBLK, DH), jnp.float32) for _ in range(NR)]
        + [pltpu.SemaphoreType.DMA,
           pltpu.SemaphoreType.DMA((NX,)),
           pltpu.SemaphoreType.DMA((NR,)),
           pltpu.SemaphoreType.DMA((NR,)),
           pltpu.VMEM_SHARED((N + 8, DH), jnp.float32)]
    ),
)


def _ep_body(d0_ref, d1_ref, b_ref, a0_ref, a1_ref, o_ref):
    dinv = lax.rsqrt(d0_ref[...] + d1_ref[...] + 1.0)
    o_ref[:, :DH] = a0_ref[...] * dinv + b_ref[:, :DH]
    o_ref[:, DH:] = a1_ref[...] * dinv + b_ref[:, DH:]


_ep_call = pl.pallas_call(
    _ep_body,
    grid=(N // R,),
    in_specs=[
        pl.BlockSpec((R, 1), lambda i: (i, 0)),
        pl.BlockSpec((R, 1), lambda i: (i, 0)),
        pl.BlockSpec((1, D), lambda i: (0, 0)),
        pl.BlockSpec((R, DH), lambda i: (i, 0)),
        pl.BlockSpec((R, DH), lambda i: (i, 0)),
    ],
    out_specs=pl.BlockSpec((R, D), lambda i: (i, 0)),
    out_shape=jax.ShapeDtypeStruct((N, D), jnp.float32),
)


def kernel(x, edge_index, W, b):
    deg0, deg1 = _deg_call(edge_index)
    d0 = deg0[:N].reshape(N, 1)
    d1 = deg1[:N].reshape(N, 1)
    g0, g1 = _mm_call(d0, d1, x, W)
    pad = EPAD - E
    src1d = jnp.concatenate([edge_index[0], jnp.zeros((pad,), jnp.int32)])
    dst1d = jnp.concatenate([edge_index[1], jnp.full((pad,), N, jnp.int32)])
    a0, a1 = _edge_call(g0, g1, src1d, dst1d)
    return _ep_call(d0, d1, b.reshape(1, D), a0, a1)


# speedup vs baseline: 2.1000x; 2.1000x over previous
import functools

import jax
import jax.numpy as jnp
from jax import lax
from jax.experimental import pallas as pl
from jax.experimental.pallas import tpu as pltpu
from jax.experimental.pallas import tpu_sc as plsc

N = 10000
E = 160000
D = 256
DH = 128
EBLK = 128
NBLKS = E // EBLK
GBLK = 128
EPAD = 172032
NB = EPAD // GBLK // 16
NR = 3
NX = 6
TRASH = 40
NC, NS = 2, 16
HIST_N = 10240
HSLC = HIST_N // NS
RPT = 632
RPT_LAST = N - (NS - 1) * RPT
R = 1000


def _sc_mesh():
    return plsc.VectorSubcoreMesh(core_axis_name="c", subcore_axis_name="s")


def _deg_body(edge_hbm, deg0_hbm, deg1_hbm, dst_v, ones_v, zeros_v, hist_sh):
    c = lax.axis_index("c")
    s = lax.axis_index("s")

    for j in range(EBLK // 16):
        ones_v[pl.ds(j * 16, 16)] = jnp.ones((16,), jnp.float32)
    for j in range(HSLC // 16):
        zeros_v[pl.ds(j * 16, 16)] = jnp.zeros((16,), jnp.float32)

    pltpu.sync_copy(zeros_v, hist_sh.at[pl.ds(s * HSLC, HSLC)])
    plsc.subcore_barrier()

    w = c * NS + s

    @pl.loop(0, (NBLKS + NC * NS - 1) // (NC * NS))
    def _edge_blocks(i):
        bi = w + i * NC * NS

        @pl.when(bi < NBLKS)
        def _():
            pltpu.sync_copy(edge_hbm.at[1, pl.ds(bi * EBLK, EBLK)], dst_v)
            pltpu.sync_copy(ones_v, hist_sh.at[dst_v], add=True)

    plsc.subcore_barrier()

    @pl.when(c == 0)
    def _():
        pltpu.sync_copy(hist_sh.at[pl.ds(s * HSLC, HSLC)],
                        deg0_hbm.at[pl.ds(s * HSLC, HSLC)])

    @pl.when(c == 1)
    def _():
        pltpu.sync_copy(hist_sh.at[pl.ds(s * HSLC, HSLC)],
                        deg1_hbm.at[pl.ds(s * HSLC, HSLC)])


_deg_call = pl.kernel(
    _deg_body,
    out_type=(jax.ShapeDtypeStruct((HIST_N,), jnp.float32),
              jax.ShapeDtypeStruct((HIST_N,), jnp.float32)),
    mesh=_sc_mesh(),
    scratch_types=[
        pltpu.VMEM((EBLK,), jnp.int32),
        pltpu.VMEM((EBLK,), jnp.float32),
        pltpu.VMEM((HSLC,), jnp.float32),
        pltpu.VMEM_SHARED((HIST_N,), jnp.float32),
    ],
)


def _mm_body(d0_ref, d1_ref, x_ref, w_ref, g0_ref, g1_ref):
    deg = d0_ref[...] + d1_ref[...] + 1.0
    dinv = lax.rsqrt(deg)
    h = jnp.dot(x_ref[...], w_ref[...], preferred_element_type=jnp.float32)
    g = h * dinv
    g0_ref[...] = g[:, :DH]
    g1_ref[...] = g[:, DH:]


_mm_call = pl.pallas_call(
    _mm_body,
    grid=(N // R,),
    in_specs=[
        pl.BlockSpec((R, 1), lambda i: (i, 0)),
        pl.BlockSpec((R, 1), lambda i: (i, 0)),
        pl.BlockSpec((R, D), lambda i: (i, 0)),
        pl.BlockSpec((D, D), lambda i: (0, 0)),
    ],
    out_specs=[
        pl.BlockSpec((R, DH), lambda i: (i, 0)),
        pl.BlockSpec((R, DH), lambda i: (i, 0)),
    ],
    out_shape=[
        jax.ShapeDtypeStruct((N, DH), jnp.float32),
        jax.ShapeDtypeStruct((N, DH), jnp.float32),
    ],
)


def _edge_body(*a):
    g0_hbm, g1_hbm, src1d, dst1d, a0_hbm, a1_hbm = a[:6]
    sb = a[6:6 + NX]
    db = a[6 + NX:6 + 2 * NX]
    rows = a[6 + 2 * NX:6 + 2 * NX + NR]
    isem, xsem, gsem, ssem, acc_sh = a[6 + 2 * NX + NR:]

    c = lax.axis_index("c")
    s = lax.axis_index("s")

    def idx_copy(base, j, slot, make_only):
        mk = pltpu.make_async_copy if make_only else \
            lambda sr, dr, sm: pltpu.async_copy(sr, dr, sm)
        ds_ = pl.ds((base + j) * GBLK, GBLK)
        return (mk(src1d.at[ds_], sb[slot], xsem.at[slot]),
                mk(dst1d.at[ds_], db[slot], xsem.at[slot]))

    def work(g_hbm, o_hbm):
        base = s * NB

        @pl.when(s < NS - 1)
        def _():
            pltpu.async_copy(g_hbm.at[pl.ds(s * RPT, RPT)],
                             acc_sh.at[pl.ds(s * RPT, RPT)], isem)

        @pl.when(s == NS - 1)
        def _():
            pltpu.async_copy(g_hbm.at[pl.ds((NS - 1) * RPT, RPT_LAST)],
                             acc_sh.at[pl.ds((NS - 1) * RPT, RPT_LAST)], isem)

        for j in range(4):
            idx_copy(base, j, j, False)
        for j in range(2):
            d1, d2 = idx_copy(base, j, j, True)
            d1.wait()
            d2.wait()
            pltpu.async_copy(g_hbm.at[sb[j]], rows[j], gsem.at[j])

        @pl.when(s < NS - 1)
        def _():
            pltpu.make_async_copy(g_hbm.at[pl.ds(s * RPT, RPT)],
                                  acc_sh.at[pl.ds(s * RPT, RPT)], isem).wait()

        @pl.when(s == NS - 1)
        def _():
            pltpu.make_async_copy(
                g_hbm.at[pl.ds((NS - 1) * RPT, RPT_LAST)],
                acc_sh.at[pl.ds((NS - 1) * RPT, RPT_LAST)], isem).wait()

        plsc.subcore_barrier()

        @pl.loop(0, NB, step=NX)
        def _blocks(o):
            for bs in range(NX):
                i = o + bs
                r = bs % NR
                r2, x2 = (bs + 2) % NR, (bs + 2) % NX
                x4 = (bs + 4) % NX
                x5 = (bs + 5) % NX

                pltpu.make_async_copy(g_hbm.at[sb[bs]], rows[r],
                                      gsem.at[r]).wait()
                pltpu.async_copy(rows[r], acc_sh.at[db[bs]],
                                 ssem.at[r], add=True)

                @pl.when(i + 2 < NB)
                def _():
                    d1, d2 = idx_copy(base, i + 2, x2, True)
                    d1.wait()
                    d2.wait()

                    @pl.when(i >= 1)
                    def _():
                        pltpu.make_async_copy(
                            rows[r2], acc_sh.at[db[x5]],
                            ssem.at[r2]).wait()

                    pltpu.async_copy(g_hbm.at[sb[x2]], rows[r2],
                                     gsem.at[r2])

                @pl.when(i + 4 < NB)
                def _():
                    idx_copy(base, i + 4, x4, False)

        for t in range(NR):
            i = NB - NR + t
            pltpu.make_async_copy(rows[i % NR], acc_sh.at[db[i % NX]],
                                  ssem.at[i % NR]).wait()

        plsc.subcore_barrier()

        @pl.when(s < NS - 1)
        def _():
            pltpu.sync_copy(acc_sh.at[pl.ds(s * RPT, RPT)],
                            o_hbm.at[pl.ds(s * RPT, RPT)])

        @pl.when(s == NS - 1)
        def _():
            pltpu.sync_copy(acc_sh.at[pl.ds((NS - 1) * RPT, RPT_LAST)],
                            o_hbm.at[pl.ds((NS - 1) * RPT, RPT_LAST)])

    @pl.when(c == 0)
    def _():
        work(g0_hbm, a0_hbm)

    @pl.when(c == 1)
    def _():
        work(g1_hbm, a1_hbm)


_edge_call = pl.kernel(
    _edge_body,
    out_type=(jax.ShapeDtypeStruct((N, DH), jnp.float32),
              jax.ShapeDtypeStruct((N, DH), jnp.float32)),
    mesh=_sc_mesh(),
    scratch_types=(
        [pltpu.VMEM((GBLK,), jnp.int32) for _ in range(2 * NX)]
        + [pltpu.VMEM((GBLK, DH), jnp.float32) for _ in range(NR)]
        + [pltpu.SemaphoreType.DMA,
           pltpu.SemaphoreType.DMA((NX,)),
           pltpu.SemaphoreType.DMA((NR,)),
           pltpu.SemaphoreType.DMA((NR,)),
           pltpu.VMEM_SHARED((N + TRASH, DH), jnp.float32)]
    ),
)


def _ep_body(d0_ref, d1_ref, b_ref, a0_ref, a1_ref, o_ref):
    dinv = lax.rsqrt(d0_ref[...] + d1_ref[...] + 1.0)
    o_ref[:, :DH] = a0_ref[...] * dinv + b_ref[:, :DH]
    o_ref[:, DH:] = a1_ref[...] * dinv + b_ref[:, DH:]


_ep_call = pl.pallas_call(
    _ep_body,
    grid=(N // R,),
    in_specs=[
        pl.BlockSpec((R, 1), lambda i: (i, 0)),
        pl.BlockSpec((R, 1), lambda i: (i, 0)),
        pl.BlockSpec((1, D), lambda i: (0, 0)),
        pl.BlockSpec((R, DH), lambda i: (i, 0)),
        pl.BlockSpec((R, DH), lambda i: (i, 0)),
    ],
    out_specs=pl.BlockSpec((R, D), lambda i: (i, 0)),
    out_shape=jax.ShapeDtypeStruct((N, D), jnp.float32),
)


def kernel(x, edge_index, W, b):
    deg0, deg1 = _deg_call(edge_index)
    d0 = deg0[:N].reshape(N, 1)
    d1 = deg1[:N].reshape(N, 1)
    g0, g1 = _mm_call(d0, d1, x, W)
    pad = EPAD - E
    pi = jnp.arange(pad, dtype=jnp.int32)
    src1d = jnp.concatenate([edge_index[0], pi % N])
    dst1d = jnp.concatenate([edge_index[1], N + pi % TRASH])
    a0, a1 = _edge_call(g0, g1, src1d, dst1d)
    return _ep_call(d0, d1, b.reshape(1, D), a0, a1)


# speedup vs baseline: 2.3908x; 1.1385x over previous
import functools

import jax
import jax.numpy as jnp
from jax import lax
from jax.experimental import pallas as pl
from jax.experimental.pallas import tpu as pltpu
from jax.experimental.pallas import tpu_sc as plsc

N = 10000
E = 160000
D = 256
DH = 128
NC, NS = 2, 16
NW = NC * NS
EBLK = 256
NBD = E // EBLK
ITD = (NBD + NW - 1) // NW
NXD = 4
GBLK = 128
NBT = E // GBLK
NB = NBT // NS
NTAIL = NBT - NS * NB
NR = 3
NX = 6
HIST_N = 10240
HSLC = HIST_N // NS
RPT = 632
RPT_LAST = N - (NS - 1) * RPT
R = 1000


def _sc_mesh():
    return plsc.VectorSubcoreMesh(core_axis_name="c", subcore_axis_name="s")


def _deg_body(*a):
    edge_hbm, deg0_hbm, deg1_hbm = a[:3]
    dstb = a[3:3 + NXD]
    ones_v, zeros_v, trash_v, hist_sh, dsem, ssem = a[3 + NXD:]

    c = lax.axis_index("c")
    s = lax.axis_index("s")

    for j in range(EBLK // 16):
        ones_v[pl.ds(j * 16, 16)] = jnp.ones((16,), jnp.float32)
        trash_v[pl.ds(j * 16, 16)] = jnp.full((16,), N + j, jnp.int32)
    for j in range(HSLC // 16):
        zeros_v[pl.ds(j * 16, 16)] = jnp.zeros((16,), jnp.float32)

    pltpu.sync_copy(zeros_v, hist_sh.at[pl.ds(s * HSLC, HSLC)])
    plsc.subcore_barrier()

    w = c * NS + s

    def idx_ds(i):
        bc = jnp.minimum(w + i * NW, NBD - 1)
        return pl.ds(bc * EBLK, EBLK)

    for j in range(2):
        pltpu.async_copy(edge_hbm.at[1, idx_ds(j)], dstb[j], dsem.at[j])

    @pl.loop(0, ITD, step=NXD)
    def _deg_blocks(o):
        for bs in range(NXD):
            i = o + bs
            j2 = (bs + 2) % NXD

            pltpu.make_async_copy(edge_hbm.at[1, idx_ds(i)], dstb[bs],
                                  dsem.at[bs]).wait()

            @pl.when(w + i * NW < NBD)
            def _():
                pltpu.async_copy(ones_v, hist_sh.at[dstb[bs]],
                                 ssem.at[bs], add=True)

            @pl.when(w + i * NW >= NBD)
            def _():
                pltpu.async_copy(ones_v, hist_sh.at[trash_v],
                                 ssem.at[bs], add=True)

            @pl.when(i + 2 < ITD)
            def _():
                @pl.when(i >= 2)
                def _():
                    pltpu.make_async_copy(ones_v, hist_sh.at[dstb[j2]],
                                          ssem.at[j2]).wait()

                pltpu.async_copy(edge_hbm.at[1, idx_ds(i + 2)], dstb[j2],
                                 dsem.at[j2])

    for t in range(NXD):
        i = ITD - NXD + t
        pltpu.make_async_copy(ones_v, hist_sh.at[dstb[i % NXD]],
                              ssem.at[i % NXD]).wait()

    plsc.subcore_barrier()

    @pl.when(c == 0)
    def _():
        pltpu.sync_copy(hist_sh.at[pl.ds(s * HSLC, HSLC)],
                        deg0_hbm.at[pl.ds(s * HSLC, HSLC)])

    @pl.when(c == 1)
    def _():
        pltpu.sync_copy(hist_sh.at[pl.ds(s * HSLC, HSLC)],
                        deg1_hbm.at[pl.ds(s * HSLC, HSLC)])


_deg_call = pl.kernel(
    _deg_body,
    out_type=(jax.ShapeDtypeStruct((HIST_N,), jnp.float32),
              jax.ShapeDtypeStruct((HIST_N,), jnp.float32)),
    mesh=_sc_mesh(),
    scratch_types=(
        [pltpu.VMEM((EBLK,), jnp.int32) for _ in range(NXD)]
        + [pltpu.VMEM((EBLK,), jnp.float32),
           pltpu.VMEM((HSLC,), jnp.float32),
           pltpu.VMEM((EBLK,), jnp.int32),
           pltpu.VMEM_SHARED((HIST_N,), jnp.float32),
           pltpu.SemaphoreType.DMA((NXD,)),
           pltpu.SemaphoreType.DMA((NXD,))]
    ),
)


def _mm_body(d0_ref, d1_ref, x_ref, w_ref, g0_ref, g1_ref):
    deg = d0_ref[...] + d1_ref[...] + 1.0
    dinv = lax.rsqrt(deg)
    h = jnp.dot(x_ref[...], w_ref[...], preferred_element_type=jnp.float32)
    g = h * dinv
    g0_ref[...] = g[:, :DH]
    g1_ref[...] = g[:, DH:]


_mm_call = pl.pallas_call(
    _mm_body,
    grid=(N // R,),
    in_specs=[
        pl.BlockSpec((R, 1), lambda i: (i, 0)),
        pl.BlockSpec((R, 1), lambda i: (i, 0)),
        pl.BlockSpec((R, D), lambda i: (i, 0)),
        pl.BlockSpec((D, D), lambda i: (0, 0)),
    ],
    out_specs=[
        pl.BlockSpec((R, DH), lambda i: (i, 0)),
        pl.BlockSpec((R, DH), lambda i: (i, 0)),
    ],
    out_shape=[
        jax.ShapeDtypeStruct((N, DH), jnp.float32),
        jax.ShapeDtypeStruct((N, DH), jnp.float32),
    ],
)


def _edge_body(*a):
    g0_hbm, g1_hbm, e_hbm, a0_hbm, a1_hbm = a[:5]
    sb = a[5:5 + NX]
    db = a[5 + NX:5 + 2 * NX]
    rows = a[5 + 2 * NX:5 + 2 * NX + NR]
    isem, xsem, gsem, ssem, acc_sh = a[5 + 2 * NX + NR:]

    c = lax.axis_index("c")
    s = lax.axis_index("s")

    def idx_copy(base, j, slot, make_only):
        mk = pltpu.make_async_copy if make_only else \
            lambda sr, dr, sm: pltpu.async_copy(sr, dr, sm)
        ds_ = pl.ds((base + j) * GBLK, GBLK)
        return (mk(e_hbm.at[0, ds_], sb[slot], xsem.at[slot]),
                mk(e_hbm.at[1, ds_], db[slot], xsem.at[slot]))

    def work(g_hbm, o_hbm):
        base = s * NB

        @pl.when(s < NS - 1)
        def _():
            pltpu.async_copy(g_hbm.at[pl.ds(s * RPT, RPT)],
                             acc_sh.at[pl.ds(s * RPT, RPT)], isem)

        @pl.when(s == NS - 1)
        def _():
            pltpu.async_copy(g_hbm.at[pl.ds((NS - 1) * RPT, RPT_LAST)],
                             acc_sh.at[pl.ds((NS - 1) * RPT, RPT_LAST)], isem)

        for j in range(4):
            idx_copy(base, j, j, False)
        for j in range(2):
            d1, d2 = idx_copy(base, j, j, True)
            d1.wait()
            d2.wait()
            pltpu.async_copy(g_hbm.at[sb[j]], rows[j], gsem.at[j])

        @pl.when(s < NS - 1)
        def _():
            pltpu.make_async_copy(g_hbm.at[pl.ds(s * RPT, RPT)],
                                  acc_sh.at[pl.ds(s * RPT, RPT)], isem).wait()

        @pl.when(s == NS - 1)
        def _():
            pltpu.make_async_copy(
                g_hbm.at[pl.ds((NS - 1) * RPT, RPT_LAST)],
                acc_sh.at[pl.ds((NS - 1) * RPT, RPT_LAST)], isem).wait()

        plsc.subcore_barrier()

        @pl.loop(0, NB, step=NX)
        def _blocks(o):
            for bs in range(NX):
                i = o + bs
                r = bs % NR
                r2, x2 = (bs + 2) % NR, (bs + 2) % NX
                x4 = (bs + 4) % NX
                x5 = (bs + 5) % NX

                pltpu.make_async_copy(g_hbm.at[sb[bs]], rows[r],
                                      gsem.at[r]).wait()
                pltpu.async_copy(rows[r], acc_sh.at[db[bs]],
                                 ssem.at[r], add=True)

                @pl.when(i + 2 < NB)
                def _():
                    d1, d2 = idx_copy(base, i + 2, x2, True)
                    d1.wait()
                    d2.wait()

                    @pl.when(i >= 1)
                    def _():
                        pltpu.make_async_copy(
                            rows[r2], acc_sh.at[db[x5]],
                            ssem.at[r2]).wait()

                    pltpu.async_copy(g_hbm.at[sb[x2]], rows[r2],
                                     gsem.at[r2])

                @pl.when(i + 4 < NB)
                def _():
                    idx_copy(base, i + 4, x4, False)

        for t in range(NR):
            i = NB - NR + t
            pltpu.make_async_copy(rows[i % NR], acc_sh.at[db[i % NX]],
                                  ssem.at[i % NR]).wait()

        @pl.when(s < NTAIL)
        def _():
            ds_ = pl.ds((NS * NB) * GBLK + s * GBLK, GBLK)
            pltpu.sync_copy(e_hbm.at[0, ds_], sb[0])
            pltpu.sync_copy(e_hbm.at[1, ds_], db[0])
            pltpu.async_copy(g_hbm.at[sb[0]], rows[0], gsem.at[0])
            pltpu.make_async_copy(g_hbm.at[sb[0]], rows[0],
                                  gsem.at[0]).wait()
            pltpu.sync_copy(rows[0], acc_sh.at[db[0]], add=True)

        plsc.subcore_barrier()

        @pl.when(s < NS - 1)
        def _():
            pltpu.sync_copy(acc_sh.at[pl.ds(s * RPT, RPT)],
                            o_hbm.at[pl.ds(s * RPT, RPT)])

        @pl.when(s == NS - 1)
        def _():
            pltpu.sync_copy(acc_sh.at[pl.ds((NS - 1) * RPT, RPT_LAST)],
                            o_hbm.at[pl.ds((NS - 1) * RPT, RPT_LAST)])

    @pl.when(c == 0)
    def _():
        work(g0_hbm, a0_hbm)

    @pl.when(c == 1)
    def _():
        work(g1_hbm, a1_hbm)


_edge_call = pl.kernel(
    _edge_body,
    out_type=(jax.ShapeDtypeStruct((N, DH), jnp.float32),
              jax.ShapeDtypeStruct((N, DH), jnp.float32)),
    mesh=_sc_mesh(),
    scratch_types=(
        [pltpu.VMEM((GBLK,), jnp.int32) for _ in range(2 * NX)]
        + [pltpu.VMEM((GBLK, DH), jnp.float32) for _ in range(NR)]
        + [pltpu.SemaphoreType.DMA,
           pltpu.SemaphoreType.DMA((NX,)),
           pltpu.SemaphoreType.DMA((NR,)),
           pltpu.SemaphoreType.DMA((NR,)),
           pltpu.VMEM_SHARED((N, DH), jnp.float32)]
    ),
)


def _ep_body(d0_ref, d1_ref, b_ref, a0_ref, a1_ref, o_ref):
    dinv = lax.rsqrt(d0_ref[...] + d1_ref[...] + 1.0)
    o_ref[:, :DH] = a0_ref[...] * dinv + b_ref[:, :DH]
    o_ref[:, DH:] = a1_ref[...] * dinv + b_ref[:, DH:]


_ep_call = pl.pallas_call(
    _ep_body,
    grid=(N // R,),
    in_specs=[
        pl.BlockSpec((R, 1), lambda i: (i, 0)),
        pl.BlockSpec((R, 1), lambda i: (i, 0)),
        pl.BlockSpec((1, D), lambda i: (0, 0)),
        pl.BlockSpec((R, DH), lambda i: (i, 0)),
        pl.BlockSpec((R, DH), lambda i: (i, 0)),
    ],
    out_specs=pl.BlockSpec((R, D), lambda i: (i, 0)),
    out_shape=jax.ShapeDtypeStruct((N, D), jnp.float32),
)


def kernel(x, edge_index, W, b):
    deg0, deg1 = _deg_call(edge_index)
    d0 = deg0[:N].reshape(N, 1)
    d1 = deg1[:N].reshape(N, 1)
    g0, g1 = _mm_call(d0, d1, x, W)
    a0, a1 = _edge_call(g0, g1, edge_index)
    return _ep_call(d0, d1, b.reshape(1, D), a0, a1)


# speedup vs baseline: 2.4511x; 1.0252x over previous
import functools

import jax
import jax.numpy as jnp
from jax import lax
from jax.experimental import pallas as pl
from jax.experimental.pallas import tpu as pltpu
from jax.experimental.pallas import tpu_sc as plsc

N = 10000
E = 160000
D = 256
DH = 128
NC, NS = 2, 16
NW = NC * NS
EBLK = 256
NBD = E // EBLK
ITD = (NBD + NW - 1) // NW
NXD = 4
GBLK = 128
NBT = E // GBLK
NB = NBT // NS
NTAIL = NBT - NS * NB
NR = 3
NX = 6
HIST_N = 10240
HSLC = HIST_N // NS
RPT = 632
RPT_LAST = N - (NS - 1) * RPT
R = 2000


def _sc_mesh():
    return plsc.VectorSubcoreMesh(core_axis_name="c", subcore_axis_name="s")


def _deg_body(*a):
    edge_hbm, deg0_hbm, deg1_hbm = a[:3]
    dstb = a[3:3 + NXD]
    ones_v, zeros_v, trash_v, hist_sh, dsem, ssem = a[3 + NXD:]

    c = lax.axis_index("c")
    s = lax.axis_index("s")

    for j in range(EBLK // 16):
        ones_v[pl.ds(j * 16, 16)] = jnp.ones((16,), jnp.float32)
        trash_v[pl.ds(j * 16, 16)] = jnp.full((16,), N + j, jnp.int32)
    for j in range(HSLC // 16):
        zeros_v[pl.ds(j * 16, 16)] = jnp.zeros((16,), jnp.float32)

    pltpu.sync_copy(zeros_v, hist_sh.at[pl.ds(s * HSLC, HSLC)])
    plsc.subcore_barrier()

    w = c * NS + s

    def idx_ds(i):
        bc = jnp.minimum(w + i * NW, NBD - 1)
        return pl.ds(bc * EBLK, EBLK)

    for j in range(2):
        pltpu.async_copy(edge_hbm.at[1, idx_ds(j)], dstb[j], dsem.at[j])

    @pl.loop(0, ITD, step=NXD)
    def _deg_blocks(o):
        for bs in range(NXD):
            i = o + bs
            j2 = (bs + 2) % NXD

            pltpu.make_async_copy(edge_hbm.at[1, idx_ds(i)], dstb[bs],
                                  dsem.at[bs]).wait()

            @pl.when(w + i * NW < NBD)
            def _():
                pltpu.async_copy(ones_v, hist_sh.at[dstb[bs]],
                                 ssem.at[bs], add=True)

            @pl.when(w + i * NW >= NBD)
            def _():
                pltpu.async_copy(ones_v, hist_sh.at[trash_v],
                                 ssem.at[bs], add=True)

            @pl.when(i + 2 < ITD)
            def _():
                @pl.when(i >= 2)
                def _():
                    pltpu.make_async_copy(ones_v, hist_sh.at[dstb[j2]],
                                          ssem.at[j2]).wait()

                pltpu.async_copy(edge_hbm.at[1, idx_ds(i + 2)], dstb[j2],
                                 dsem.at[j2])

    for t in range(NXD):
        i = ITD - NXD + t
        pltpu.make_async_copy(ones_v, hist_sh.at[dstb[i % NXD]],
                              ssem.at[i % NXD]).wait()

    plsc.subcore_barrier()

    @pl.when(c == 0)
    def _():
        pltpu.sync_copy(hist_sh.at[pl.ds(s * HSLC, HSLC)],
                        deg0_hbm.at[pl.ds(s * HSLC, HSLC)])

    @pl.when(c == 1)
    def _():
        pltpu.sync_copy(hist_sh.at[pl.ds(s * HSLC, HSLC)],
                        deg1_hbm.at[pl.ds(s * HSLC, HSLC)])


_deg_call = pl.kernel(
    _deg_body,
    out_type=(jax.ShapeDtypeStruct((HIST_N,), jnp.float32),
              jax.ShapeDtypeStruct((HIST_N,), jnp.float32)),
    mesh=_sc_mesh(),
    scratch_types=(
        [pltpu.VMEM((EBLK,), jnp.int32) for _ in range(NXD)]
        + [pltpu.VMEM((EBLK,), jnp.float32),
           pltpu.VMEM((HSLC,), jnp.float32),
           pltpu.VMEM((EBLK,), jnp.int32),
           pltpu.VMEM_SHARED((HIST_N,), jnp.float32),
           pltpu.SemaphoreType.DMA((NXD,)),
           pltpu.SemaphoreType.DMA((NXD,))]
    ),
)


def _mm_body(d0_ref, d1_ref, x_ref, w_ref, g0_ref, g1_ref):
    deg = d0_ref[...] + d1_ref[...] + 1.0
    dinv = lax.rsqrt(deg)
    h = jnp.dot(x_ref[...], w_ref[...], preferred_element_type=jnp.float32)
    g = h * dinv
    g0_ref[...] = g[:, :DH]
    g1_ref[...] = g[:, DH:]


_mm_call = pl.pallas_call(
    _mm_body,
    grid=(N // R,),
    in_specs=[
        pl.BlockSpec((R, 1), lambda i: (i, 0)),
        pl.BlockSpec((R, 1), lambda i: (i, 0)),
        pl.BlockSpec((R, D), lambda i: (i, 0)),
        pl.BlockSpec((D, D), lambda i: (0, 0)),
    ],
    out_specs=[
        pl.BlockSpec((R, DH), lambda i: (i, 0)),
        pl.BlockSpec((R, DH), lambda i: (i, 0)),
    ],
    out_shape=[
        jax.ShapeDtypeStruct((N, DH), jnp.float32),
        jax.ShapeDtypeStruct((N, DH), jnp.float32),
    ],
)


def _edge_body(*a):
    g0_hbm, g1_hbm, e_hbm, a0_hbm, a1_hbm = a[:5]
    sb = a[5:5 + NX]
    db = a[5 + NX:5 + 2 * NX]
    rows = a[5 + 2 * NX:5 + 2 * NX + NR]
    isem, xsem, gsem, ssem, acc_sh = a[5 + 2 * NX + NR:]

    c = lax.axis_index("c")
    s = lax.axis_index("s")

    def idx_copy(base, j, slot, make_only):
        mk = pltpu.make_async_copy if make_only else \
            lambda sr, dr, sm: pltpu.async_copy(sr, dr, sm)
        ds_ = pl.ds((base + j) * GBLK, GBLK)
        return (mk(e_hbm.at[0, ds_], sb[slot], xsem.at[slot]),
                mk(e_hbm.at[1, ds_], db[slot], xsem.at[slot]))

    def work(g_hbm, o_hbm):
        base = s * NB

        @pl.when(s < NS - 1)
        def _():
            pltpu.async_copy(g_hbm.at[pl.ds(s * RPT, RPT)],
                             acc_sh.at[pl.ds(s * RPT, RPT)], isem)

        @pl.when(s == NS - 1)
        def _():
            pltpu.async_copy(g_hbm.at[pl.ds((NS - 1) * RPT, RPT_LAST)],
                             acc_sh.at[pl.ds((NS - 1) * RPT, RPT_LAST)], isem)

        for j in range(4):
            idx_copy(base, j, j, False)
        for j in range(2):
            d1, d2 = idx_copy(base, j, j, True)
            d1.wait()
            d2.wait()
            pltpu.async_copy(g_hbm.at[sb[j]], rows[j], gsem.at[j])

        @pl.when(s < NS - 1)
        def _():
            pltpu.make_async_copy(g_hbm.at[pl.ds(s * RPT, RPT)],
                                  acc_sh.at[pl.ds(s * RPT, RPT)], isem).wait()

        @pl.when(s == NS - 1)
        def _():
            pltpu.make_async_copy(
                g_hbm.at[pl.ds((NS - 1) * RPT, RPT_LAST)],
                acc_sh.at[pl.ds((NS - 1) * RPT, RPT_LAST)], isem).wait()

        plsc.subcore_barrier()

        @pl.loop(0, NB, step=NX)
        def _blocks(o):
            for bs in range(NX):
                i = o + bs
                r = bs % NR
                r2, x2 = (bs + 2) % NR, (bs + 2) % NX
                x4 = (bs + 4) % NX
                x5 = (bs + 5) % NX

                pltpu.make_async_copy(g_hbm.at[sb[bs]], rows[r],
                                      gsem.at[r]).wait()
                pltpu.async_copy(rows[r], acc_sh.at[db[bs]],
                                 ssem.at[r], add=True)

                @pl.when(i + 2 < NB)
                def _():
                    d1, d2 = idx_copy(base, i + 2, x2, True)
                    d1.wait()
                    d2.wait()

                    @pl.when(i >= 1)
                    def _():
                        pltpu.make_async_copy(
                            rows[r2], acc_sh.at[db[x5]],
                            ssem.at[r2]).wait()

                    pltpu.async_copy(g_hbm.at[sb[x2]], rows[r2],
                                     gsem.at[r2])

                @pl.when(i + 4 < NB)
                def _():
                    idx_copy(base, i + 4, x4, False)

        for t in range(NR):
            i = NB - NR + t
            pltpu.make_async_copy(rows[i % NR], acc_sh.at[db[i % NX]],
                                  ssem.at[i % NR]).wait()

        @pl.when(s < NTAIL)
        def _():
            ds_ = pl.ds((NS * NB) * GBLK + s * GBLK, GBLK)
            pltpu.sync_copy(e_hbm.at[0, ds_], sb[0])
            pltpu.sync_copy(e_hbm.at[1, ds_], db[0])
            pltpu.async_copy(g_hbm.at[sb[0]], rows[0], gsem.at[0])
            pltpu.make_async_copy(g_hbm.at[sb[0]], rows[0],
                                  gsem.at[0]).wait()
            pltpu.sync_copy(rows[0], acc_sh.at[db[0]], add=True)

        plsc.subcore_barrier()

        @pl.when(s < NS - 1)
        def _():
            pltpu.sync_copy(acc_sh.at[pl.ds(s * RPT, RPT)],
                            o_hbm.at[pl.ds(s * RPT, RPT)])

        @pl.when(s == NS - 1)
        def _():
            pltpu.sync_copy(acc_sh.at[pl.ds((NS - 1) * RPT, RPT_LAST)],
                            o_hbm.at[pl.ds((NS - 1) * RPT, RPT_LAST)])

    @pl.when(c == 0)
    def _():
        work(g0_hbm, a0_hbm)

    @pl.when(c == 1)
    def _():
        work(g1_hbm, a1_hbm)


_edge_call = pl.kernel(
    _edge_body,
    out_type=(jax.ShapeDtypeStruct((N, DH), jnp.float32),
              jax.ShapeDtypeStruct((N, DH), jnp.float32)),
    mesh=_sc_mesh(),
    scratch_types=(
        [pltpu.VMEM((GBLK,), jnp.int32) for _ in range(2 * NX)]
        + [pltpu.VMEM((GBLK, DH), jnp.float32) for _ in range(NR)]
        + [pltpu.SemaphoreType.DMA,
           pltpu.SemaphoreType.DMA((NX,)),
           pltpu.SemaphoreType.DMA((NR,)),
           pltpu.SemaphoreType.DMA((NR,)),
           pltpu.VMEM_SHARED((N, DH), jnp.float32)]
    ),
)


def _ep_body(d0_ref, d1_ref, b_ref, a0_ref, a1_ref, o_ref):
    dinv = lax.rsqrt(d0_ref[...] + d1_ref[...] + 1.0)
    o_ref[:, :DH] = a0_ref[...] * dinv + b_ref[:, :DH]
    o_ref[:, DH:] = a1_ref[...] * dinv + b_ref[:, DH:]


_ep_call = pl.pallas_call(
    _ep_body,
    grid=(N // R,),
    in_specs=[
        pl.BlockSpec((R, 1), lambda i: (i, 0)),
        pl.BlockSpec((R, 1), lambda i: (i, 0)),
        pl.BlockSpec((1, D), lambda i: (0, 0)),
        pl.BlockSpec((R, DH), lambda i: (i, 0)),
        pl.BlockSpec((R, DH), lambda i: (i, 0)),
    ],
    out_specs=pl.BlockSpec((R, D), lambda i: (i, 0)),
    out_shape=jax.ShapeDtypeStruct((N, D), jnp.float32),
)


def kernel(x, edge_index, W, b):
    deg0, deg1 = _deg_call(edge_index)
    d0 = deg0[:N].reshape(N, 1)
    d1 = deg1[:N].reshape(N, 1)
    g0, g1 = _mm_call(d0, d1, x, W)
    a0, a1 = _edge_call(g0, g1, edge_index)
    return _ep_call(d0, d1, b.reshape(1, D), a0, a1)


# speedup vs baseline: 2.4842x; 1.0135x over previous
import functools

import jax
import jax.numpy as jnp
from jax import lax
from jax.experimental import pallas as pl
from jax.experimental.pallas import tpu as pltpu
from jax.experimental.pallas import tpu_sc as plsc

N = 10000
E = 160000
D = 256
DH = 128
NC, NS = 2, 16
NW = NC * NS
EBLK = 256
NBD = E // EBLK
ITD = (NBD + NW - 1) // NW
NXD = 4
GBLK = 128
NBT = E // GBLK
NB = NBT // NS
NTAIL = NBT - NS * NB
NR = 3
NX = 6
HIST_N = 10240
HSLC = HIST_N // NS
RPT = 632
RPT_LAST = N - (NS - 1) * RPT
R = 5000


def _sc_mesh():
    return plsc.VectorSubcoreMesh(core_axis_name="c", subcore_axis_name="s")


def _deg_body(*a):
    edge_hbm, deg0_hbm, deg1_hbm = a[:3]
    dstb = a[3:3 + NXD]
    ones_v, zeros_v, trash_v, hist_sh, dsem, ssem = a[3 + NXD:]

    c = lax.axis_index("c")
    s = lax.axis_index("s")

    for j in range(EBLK // 16):
        ones_v[pl.ds(j * 16, 16)] = jnp.ones((16,), jnp.float32)
        trash_v[pl.ds(j * 16, 16)] = jnp.full((16,), N + j, jnp.int32)
    for j in range(HSLC // 16):
        zeros_v[pl.ds(j * 16, 16)] = jnp.zeros((16,), jnp.float32)

    pltpu.sync_copy(zeros_v, hist_sh.at[pl.ds(s * HSLC, HSLC)])
    plsc.subcore_barrier()

    w = c * NS + s

    def idx_ds(i):
        bc = jnp.minimum(w + i * NW, NBD - 1)
        return pl.ds(bc * EBLK, EBLK)

    for j in range(2):
        pltpu.async_copy(edge_hbm.at[1, idx_ds(j)], dstb[j], dsem.at[j])

    @pl.loop(0, ITD, step=NXD)
    def _deg_blocks(o):
        for bs in range(NXD):
            i = o + bs
            j2 = (bs + 2) % NXD

            pltpu.make_async_copy(edge_hbm.at[1, idx_ds(i)], dstb[bs],
                                  dsem.at[bs]).wait()

            @pl.when(w + i * NW < NBD)
            def _():
                pltpu.async_copy(ones_v, hist_sh.at[dstb[bs]],
                                 ssem.at[bs], add=True)

            @pl.when(w + i * NW >= NBD)
            def _():
                pltpu.async_copy(ones_v, hist_sh.at[trash_v],
                                 ssem.at[bs], add=True)

            @pl.when(i + 2 < ITD)
            def _():
                @pl.when(i >= 2)
                def _():
                    pltpu.make_async_copy(ones_v, hist_sh.at[dstb[j2]],
                                          ssem.at[j2]).wait()

                pltpu.async_copy(edge_hbm.at[1, idx_ds(i + 2)], dstb[j2],
                                 dsem.at[j2])

    for t in range(NXD):
        i = ITD - NXD + t
        pltpu.make_async_copy(ones_v, hist_sh.at[dstb[i % NXD]],
                              ssem.at[i % NXD]).wait()

    plsc.subcore_barrier()

    @pl.when(c == 0)
    def _():
        pltpu.sync_copy(hist_sh.at[pl.ds(s * HSLC, HSLC)],
                        deg0_hbm.at[pl.ds(s * HSLC, HSLC)])

    @pl.when(c == 1)
    def _():
        pltpu.sync_copy(hist_sh.at[pl.ds(s * HSLC, HSLC)],
                        deg1_hbm.at[pl.ds(s * HSLC, HSLC)])


_deg_call = pl.kernel(
    _deg_body,
    out_type=(jax.ShapeDtypeStruct((HIST_N,), jnp.float32),
              jax.ShapeDtypeStruct((HIST_N,), jnp.float32)),
    mesh=_sc_mesh(),
    scratch_types=(
        [pltpu.VMEM((EBLK,), jnp.int32) for _ in range(NXD)]
        + [pltpu.VMEM((EBLK,), jnp.float32),
           pltpu.VMEM((HSLC,), jnp.float32),
           pltpu.VMEM((EBLK,), jnp.int32),
           pltpu.VMEM_SHARED((HIST_N,), jnp.float32),
           pltpu.SemaphoreType.DMA((NXD,)),
           pltpu.SemaphoreType.DMA((NXD,))]
    ),
)


def _mm_body(d0_ref, d1_ref, x_ref, w_ref, g0_ref, g1_ref):
    deg = d0_ref[...] + d1_ref[...] + 1.0
    dinv = lax.rsqrt(deg)
    h = jnp.dot(x_ref[...], w_ref[...], preferred_element_type=jnp.float32)
    g = h * dinv
    g0_ref[...] = g[:, :DH]
    g1_ref[...] = g[:, DH:]


_mm_call = pl.pallas_call(
    _mm_body,
    grid=(N // R,),
    in_specs=[
        pl.BlockSpec((R, 1), lambda i: (i, 0)),
        pl.BlockSpec((R, 1), lambda i: (i, 0)),
        pl.BlockSpec((R, D), lambda i: (i, 0)),
        pl.BlockSpec((D, D), lambda i: (0, 0)),
    ],
    out_specs=[
        pl.BlockSpec((R, DH), lambda i: (i, 0)),
        pl.BlockSpec((R, DH), lambda i: (i, 0)),
    ],
    out_shape=[
        jax.ShapeDtypeStruct((N, DH), jnp.float32),
        jax.ShapeDtypeStruct((N, DH), jnp.float32),
    ],
)


def _edge_body(*a):
    g0_hbm, g1_hbm, e_hbm, a0_hbm, a1_hbm = a[:5]
    sb = a[5:5 + NX]
    db = a[5 + NX:5 + 2 * NX]
    rows = a[5 + 2 * NX:5 + 2 * NX + NR]
    isem, xsem, gsem, ssem, acc_sh = a[5 + 2 * NX + NR:]

    c = lax.axis_index("c")
    s = lax.axis_index("s")

    def idx_copy(base, j, slot, make_only):
        mk = pltpu.make_async_copy if make_only else \
            lambda sr, dr, sm: pltpu.async_copy(sr, dr, sm)
        ds_ = pl.ds((base + j) * GBLK, GBLK)
        return (mk(e_hbm.at[0, ds_], sb[slot], xsem.at[slot]),
                mk(e_hbm.at[1, ds_], db[slot], xsem.at[slot]))

    def work(g_hbm, o_hbm):
        base = s * NB

        @pl.when(s < NS - 1)
        def _():
            pltpu.async_copy(g_hbm.at[pl.ds(s * RPT, RPT)],
                             acc_sh.at[pl.ds(s * RPT, RPT)], isem)

        @pl.when(s == NS - 1)
        def _():
            pltpu.async_copy(g_hbm.at[pl.ds((NS - 1) * RPT, RPT_LAST)],
                             acc_sh.at[pl.ds((NS - 1) * RPT, RPT_LAST)], isem)

        for j in range(4):
            idx_copy(base, j, j, False)
        for j in range(2):
            d1, d2 = idx_copy(base, j, j, True)
            d1.wait()
            d2.wait()
            pltpu.async_copy(g_hbm.at[sb[j]], rows[j], gsem.at[j])

        @pl.when(s < NS - 1)
        def _():
            pltpu.make_async_copy(g_hbm.at[pl.ds(s * RPT, RPT)],
                                  acc_sh.at[pl.ds(s * RPT, RPT)], isem).wait()

        @pl.when(s == NS - 1)
        def _():
            pltpu.make_async_copy(
                g_hbm.at[pl.ds((NS - 1) * RPT, RPT_LAST)],
                acc_sh.at[pl.ds((NS - 1) * RPT, RPT_LAST)], isem).wait()

        plsc.subcore_barrier()

        @pl.loop(0, NB, step=NX)
        def _blocks(o):
            for bs in range(NX):
                i = o + bs
                r = bs % NR
                r2, x2 = (bs + 2) % NR, (bs + 2) % NX
                x4 = (bs + 4) % NX
                x5 = (bs + 5) % NX

                pltpu.make_async_copy(g_hbm.at[sb[bs]], rows[r],
                                      gsem.at[r]).wait()
                pltpu.async_copy(rows[r], acc_sh.at[db[bs]],
                                 ssem.at[r], add=True)

                @pl.when(i + 2 < NB)
                def _():
                    d1, d2 = idx_copy(base, i + 2, x2, True)
                    d1.wait()
                    d2.wait()

                    @pl.when(i >= 1)
                    def _():
                        pltpu.make_async_copy(
                            rows[r2], acc_sh.at[db[x5]],
                            ssem.at[r2]).wait()

                    pltpu.async_copy(g_hbm.at[sb[x2]], rows[r2],
                                     gsem.at[r2])

                @pl.when(i + 4 < NB)
                def _():
                    idx_copy(base, i + 4, x4, False)

        for t in range(NR):
            i = NB - NR + t
            pltpu.make_async_copy(rows[i % NR], acc_sh.at[db[i % NX]],
                                  ssem.at[i % NR]).wait()

        @pl.when(s < NTAIL)
        def _():
            ds_ = pl.ds((NS * NB) * GBLK + s * GBLK, GBLK)
            pltpu.sync_copy(e_hbm.at[0, ds_], sb[0])
            pltpu.sync_copy(e_hbm.at[1, ds_], db[0])
            pltpu.async_copy(g_hbm.at[sb[0]], rows[0], gsem.at[0])
            pltpu.make_async_copy(g_hbm.at[sb[0]], rows[0],
                                  gsem.at[0]).wait()
            pltpu.sync_copy(rows[0], acc_sh.at[db[0]], add=True)

        plsc.subcore_barrier()

        @pl.when(s < NS - 1)
        def _():
            pltpu.sync_copy(acc_sh.at[pl.ds(s * RPT, RPT)],
                            o_hbm.at[pl.ds(s * RPT, RPT)])

        @pl.when(s == NS - 1)
        def _():
            pltpu.sync_copy(acc_sh.at[pl.ds((NS - 1) * RPT, RPT_LAST)],
                            o_hbm.at[pl.ds((NS - 1) * RPT, RPT_LAST)])

    @pl.when(c == 0)
    def _():
        work(g0_hbm, a0_hbm)

    @pl.when(c == 1)
    def _():
        work(g1_hbm, a1_hbm)


_edge_call = pl.kernel(
    _edge_body,
    out_type=(jax.ShapeDtypeStruct((N, DH), jnp.float32),
              jax.ShapeDtypeStruct((N, DH), jnp.float32)),
    mesh=_sc_mesh(),
    scratch_types=(
        [pltpu.VMEM((GBLK,), jnp.int32) for _ in range(2 * NX)]
        + [pltpu.VMEM((GBLK, DH), jnp.float32) for _ in range(NR)]
        + [pltpu.SemaphoreType.DMA,
           pltpu.SemaphoreType.DMA((NX,)),
           pltpu.SemaphoreType.DMA((NR,)),
           pltpu.SemaphoreType.DMA((NR,)),
           pltpu.VMEM_SHARED((N, DH), jnp.float32)]
    ),
)


def _ep_body(d0_ref, d1_ref, b_ref, a0_ref, a1_ref, o_ref):
    dinv = lax.rsqrt(d0_ref[...] + d1_ref[...] + 1.0)
    o_ref[:, :DH] = a0_ref[...] * dinv + b_ref[:, :DH]
    o_ref[:, DH:] = a1_ref[...] * dinv + b_ref[:, DH:]


_ep_call = pl.pallas_call(
    _ep_body,
    grid=(N // R,),
    in_specs=[
        pl.BlockSpec((R, 1), lambda i: (i, 0)),
        pl.BlockSpec((R, 1), lambda i: (i, 0)),
        pl.BlockSpec((1, D), lambda i: (0, 0)),
        pl.BlockSpec((R, DH), lambda i: (i, 0)),
        pl.BlockSpec((R, DH), lambda i: (i, 0)),
    ],
    out_specs=pl.BlockSpec((R, D), lambda i: (i, 0)),
    out_shape=jax.ShapeDtypeStruct((N, D), jnp.float32),
)


def kernel(x, edge_index, W, b):
    deg0, deg1 = _deg_call(edge_index)
    d0 = deg0[:N].reshape(N, 1)
    d1 = deg1[:N].reshape(N, 1)
    g0, g1 = _mm_call(d0, d1, x, W)
    a0, a1 = _edge_call(g0, g1, edge_index)
    return _ep_call(d0, d1, b.reshape(1, D), a0, a1)
